# baseline (speedup 1.0000x reference)
import jax
import jax.numpy as jnp
from jax.experimental import pallas as pl
from jax.experimental.pallas import tpu as pltpu

_D = 120
_NBINS = 121
_POOL = 8
_HB = 2


def _img_broadcast(img):
    hb, c, w = img.shape
    parts = [
        jax.lax.broadcast_in_dim(jnp.transpose(img[h]), (w, c, _D), (0, 1))
        for h in range(hb)
    ]
    return jnp.concatenate(parts, axis=0)


def _tgt_kernel(dm_ref, img_ref, bin_ref, tgt_ref, pooled_ref):
    n = pl.program_id(1)

    @pl.when(n == 0)
    def _pool():
        dm = dm_ref[0]
        ws, hs = dm.shape
        h, w = hs // _POOL, ws // _POOL
        ra = jax.lax.broadcasted_iota(jnp.int32, (w, ws), 0)
        ca = jax.lax.broadcasted_iota(jnp.int32, (w, ws), 1)
        pool_l = (ca // _POOL == ra).astype(jnp.float32)
        rb = jax.lax.broadcasted_iota(jnp.int32, (hs, h), 0)
        cb = jax.lax.broadcasted_iota(jnp.int32, (hs, h), 1)
        pool_r = (rb // _POOL == cb).astype(jnp.float32)
        hp = jax.lax.Precision.HIGHEST
        val = jnp.dot(
            jnp.dot(pool_l, dm, precision=hp,
                    preferred_element_type=jnp.float32),
            pool_r, precision=hp, preferred_element_type=jnp.float32)
        nz = (dm != 0.0).astype(jnp.float32)
        cnt = jnp.dot(
            jnp.dot(pool_l, nz, preferred_element_type=jnp.float32),
            pool_r, preferred_element_type=jnp.float32)
        inv = 1.0 / (_POOL * _POOL)
        pooled_ref[0] = (val * inv) / (cnt * inv + 1e-10)

    img = img_ref[0]
    hb, c, w = img.shape
    p = hb * w
    img_b = _img_broadcast(img)
    bvt = jnp.concatenate(
        [jnp.transpose(bin_ref[h]) for h in range(hb)], axis=0)
    dd = jax.lax.broadcasted_iota(jnp.int32, (p, _D), 1)
    mask = dd == bvt
    mask_b = jax.lax.broadcast_in_dim(mask, (p, c, _D), (0, 2))
    tgt_ref[0] = jnp.where(mask_b, img_b, 0.0)


def _out_kernel(logits_ref, img_ref, out_ref):
    x = logits_ref[0]
    m = jnp.max(x, axis=-1, keepdims=True)
    e = jnp.exp(x - m)
    s = jnp.sum(e, axis=-1, keepdims=True)
    pv = (e / s)[:, :_D]

    img = img_ref[0]
    hb, c, w = img.shape
    p = hb * w
    img_b = _img_broadcast(img)
    pv_b = jax.lax.broadcast_in_dim(pv, (p, c, _D), (0, 2))
    out_ref[0] = img_b * pv_b


def kernel(image_features, depth_logits, depth_maps, depth_target_bin):
    B, C, H, W = image_features.shape
    N = H * W
    blk = _HB * W
    nh = -(-H // _HB)
    hp_ = nh * _HB

    logits_t = depth_logits.reshape(B, _NBINS, N).transpose(0, 2, 1)
    dm_t = depth_maps.transpose(0, 2, 1)
    img_n = image_features.transpose(0, 2, 1, 3)
    bin_p = jnp.pad(depth_target_bin, ((0, 0), (0, hp_ - H), (0, 0)))
    bin_p = bin_p.reshape(B * hp_, 1, W)

    cparams = pltpu.CompilerParams(
        dimension_semantics=("parallel", "arbitrary"),
        vmem_limit_bytes=56 * 1024 * 1024,
    )

    tgt_t, pooled_t = pl.pallas_call(
        _tgt_kernel,
        grid=(B, nh),
        in_specs=[
            pl.BlockSpec((1, W * _POOL, H * _POOL), lambda b, n: (b, 0, 0)),
            pl.BlockSpec((1, _HB, C, W), lambda b, n: (b, n, 0, 0)),
            pl.BlockSpec((_HB, 1, W), lambda b, n: (b * nh + n, 0, 0)),
        ],
        out_specs=[
            pl.BlockSpec((1, blk, C, _D), lambda b, n: (b, n, 0, 0)),
            pl.BlockSpec((1, W, H), lambda b, n: (b, 0, 0)),
        ],
        out_shape=[
            jax.ShapeDtypeStruct((B, N, C, _D), jnp.float32),
            jax.ShapeDtypeStruct((B, W, H), jnp.float32),
        ],
        compiler_params=cparams,
        name="depth_ffn_tgt",
    )(dm_t, img_n, bin_p)

    out_t = pl.pallas_call(
        _out_kernel,
        grid=(B, nh),
        in_specs=[
            pl.BlockSpec((1, blk, _NBINS), lambda b, n: (b, n, 0)),
            pl.BlockSpec((1, _HB, C, W), lambda b, n: (b, n, 0, 0)),
        ],
        out_specs=pl.BlockSpec((1, blk, C, _D), lambda b, n: (b, n, 0, 0)),
        out_shape=jax.ShapeDtypeStruct((B, N, C, _D), jnp.float32),
        compiler_params=cparams,
        name="depth_ffn_out",
    )(logits_t, img_n)

    frustum = out_t.reshape(B, H, W, C, _D).transpose(0, 3, 4, 1, 2)
    frustum_tgt = tgt_t.reshape(B, H, W, C, _D).transpose(0, 3, 4, 1, 2)
    pooled = pooled_t.transpose(0, 2, 1)
    return frustum, frustum_tgt, pooled


# speedup vs baseline: 1.2220x; 1.2220x over previous
import jax
import jax.numpy as jnp
from jax.experimental import pallas as pl
from jax.experimental.pallas import tpu as pltpu

_D = 120
_NBINS = 121
_POOL = 8
_HB = 4


def _fused_kernel(logits_ref, dm_ref, img_ref, bin_ref,
                  out_ref, tgt_ref, pooled_ref):
    n = pl.program_id(1)

    @pl.when(n == 0)
    def _pool():
        dm = dm_ref[0]
        ws, hs = dm.shape
        h, w = hs // _POOL, ws // _POOL
        ra = jax.lax.broadcasted_iota(jnp.int32, (w, ws), 0)
        ca = jax.lax.broadcasted_iota(jnp.int32, (w, ws), 1)
        pool_l = (ca // _POOL == ra).astype(jnp.float32)
        rb = jax.lax.broadcasted_iota(jnp.int32, (hs, h), 0)
        cb = jax.lax.broadcasted_iota(jnp.int32, (hs, h), 1)
        pool_r = (rb // _POOL == cb).astype(jnp.float32)
        hp = jax.lax.Precision.HIGHEST
        val = jnp.dot(
            jnp.dot(pool_l, dm, precision=hp,
                    preferred_element_type=jnp.float32),
            pool_r, precision=hp, preferred_element_type=jnp.float32)
        nz = (dm != 0.0).astype(jnp.float32)
        cnt = jnp.dot(
            jnp.dot(pool_l, nz, preferred_element_type=jnp.float32),
            pool_r, preferred_element_type=jnp.float32)
        inv = 1.0 / (_POOL * _POOL)
        pooled_ref[0] = (val * inv) / (cnt * inv + 1e-10)

    x = logits_ref[0]
    m = jnp.max(x, axis=-1, keepdims=True)
    e = jnp.exp(x - m)
    s = jnp.sum(e, axis=-1, keepdims=True)
    pv = (e / s)[:, :_D]

    img = img_ref[0]
    hb, c, w = img.shape
    p = hb * w
    img_parts = []
    bin_parts = []
    for h in range(hb):
        i_t = jnp.transpose(img[h])
        img_parts.append(jax.lax.broadcast_in_dim(i_t, (w, c, _D), (0, 1)))
        bin_parts.append(jnp.transpose(bin_ref[h]))
    img_b = jnp.concatenate(img_parts, axis=0)
    pv_b = jax.lax.broadcast_in_dim(pv, (p, c, _D), (0, 2))
    out_ref[0] = img_b * pv_b
    bvt = jnp.concatenate(bin_parts, axis=0)
    dd = jax.lax.broadcasted_iota(jnp.int32, (p, _D), 1)
    mask = dd == bvt
    mask_b = jax.lax.broadcast_in_dim(mask, (p, c, _D), (0, 2))
    tgt_ref[0] = jnp.where(mask_b, img_b, 0.0)


def kernel(image_features, depth_logits, depth_maps, depth_target_bin):
    B, C, H, W = image_features.shape
    N = H * W
    blk = _HB * W
    nh = -(-H // _HB)
    hp_ = nh * _HB

    logits_t = depth_logits.reshape(B, _NBINS, N).transpose(0, 2, 1)
    dm_t = depth_maps.transpose(0, 2, 1)
    img_n = image_features.transpose(0, 2, 1, 3)
    bin_p = jnp.pad(depth_target_bin, ((0, 0), (0, hp_ - H), (0, 0)))
    bin_p = bin_p.reshape(B * hp_, 1, W)

    out_t, tgt_t, pooled_t = pl.pallas_call(
        _fused_kernel,
        grid=(B, nh),
        in_specs=[
            pl.BlockSpec((1, blk, _NBINS), lambda b, n: (b, n, 0)),
            pl.BlockSpec((1, W * _POOL, H * _POOL), lambda b, n: (b, 0, 0)),
            pl.BlockSpec((1, _HB, C, W), lambda b, n: (b, n, 0, 0)),
            pl.BlockSpec((_HB, 1, W), lambda b, n: (b * nh + n, 0, 0)),
        ],
        out_specs=[
            pl.BlockSpec((1, blk, C, _D), lambda b, n: (b, n, 0, 0)),
            pl.BlockSpec((1, blk, C, _D), lambda b, n: (b, n, 0, 0)),
            pl.BlockSpec((1, W, H), lambda b, n: (b, 0, 0)),
        ],
        out_shape=[
            jax.ShapeDtypeStruct((B, N, C, _D), jnp.float32),
            jax.ShapeDtypeStruct((B, N, C, _D), jnp.float32),
            jax.ShapeDtypeStruct((B, W, H), jnp.float32),
        ],
        compiler_params=pltpu.CompilerParams(
            dimension_semantics=("parallel", "arbitrary"),
            vmem_limit_bytes=56 * 1024 * 1024,
        ),
        name="depth_ffn_fused",
    )(logits_t, dm_t, img_n, bin_p)

    frustum = out_t.reshape(B, H, W, C, _D).transpose(0, 3, 4, 1, 2)
    frustum_tgt = tgt_t.reshape(B, H, W, C, _D).transpose(0, 3, 4, 1, 2)
    pooled = pooled_t.transpose(0, 2, 1)
    return frustum, frustum_tgt, pooled


# speedup vs baseline: 1.2390x; 1.0139x over previous
import jax
import jax.numpy as jnp
from jax.experimental import pallas as pl
from jax.experimental.pallas import tpu as pltpu

_D = 120
_NBINS = 121
_POOL = 8
_HB = 2


def _fused_kernel(logits_ref, dm_ref, img_ref, bin_ref,
                  out_ref, tgt_ref, pooled_ref):
    n = pl.program_id(1)

    @pl.when(n == 0)
    def _pool():
        dm = dm_ref[0]
        ws, hs = dm.shape
        h, w = hs // _POOL, ws // _POOL
        ra = jax.lax.broadcasted_iota(jnp.int32, (w, ws), 0)
        ca = jax.lax.broadcasted_iota(jnp.int32, (w, ws), 1)
        pool_l = (ca // _POOL == ra).astype(jnp.float32)
        rb = jax.lax.broadcasted_iota(jnp.int32, (hs, h), 0)
        cb = jax.lax.broadcasted_iota(jnp.int32, (hs, h), 1)
        pool_r = (rb // _POOL == cb).astype(jnp.float32)
        hp = jax.lax.Precision.HIGHEST
        val = jnp.dot(
            jnp.dot(pool_l, dm, precision=hp,
                    preferred_element_type=jnp.float32),
            pool_r, precision=hp, preferred_element_type=jnp.float32)
        nz = (dm != 0.0).astype(jnp.float32)
        cnt = jnp.dot(
            jnp.dot(pool_l, nz, preferred_element_type=jnp.float32),
            pool_r, preferred_element_type=jnp.float32)
        inv = 1.0 / (_POOL * _POOL)
        pooled_ref[0] = (val * inv) / (cnt * inv + 1e-10)

    x = logits_ref[0]
    m = jnp.max(x, axis=-1, keepdims=True)
    e = jnp.exp(x - m)
    s = jnp.sum(e, axis=-1, keepdims=True)
    pv = (e / s)[:, :_D]

    img = img_ref[0]
    hb, c, w = img.shape
    p = hb * w
    img_parts = []
    bin_parts = []
    for h in range(hb):
        i_t = jnp.transpose(img[h])
        img_parts.append(jax.lax.broadcast_in_dim(i_t, (w, c, _D), (0, 1)))
        bin_parts.append(jnp.transpose(bin_ref[h]))
    img_b = jnp.concatenate(img_parts, axis=0)
    pv_b = jax.lax.broadcast_in_dim(pv, (p, c, _D), (0, 2))
    out_ref[0] = img_b * pv_b
    bvt = jnp.concatenate(bin_parts, axis=0)
    dd = jax.lax.broadcasted_iota(jnp.int32, (p, _D), 1)
    mask = dd == bvt
    mask_b = jax.lax.broadcast_in_dim(mask, (p, c, _D), (0, 2))
    tgt_ref[0] = jnp.where(mask_b, img_b, 0.0)


def kernel(image_features, depth_logits, depth_maps, depth_target_bin):
    B, C, H, W = image_features.shape
    N = H * W
    blk = _HB * W
    nh = -(-H // _HB)
    hp_ = nh * _HB

    logits_t = depth_logits.reshape(B, _NBINS, N).transpose(0, 2, 1)
    dm_t = depth_maps.transpose(0, 2, 1)
    img_n = image_features.transpose(0, 2, 1, 3)
    bin_p = jnp.pad(depth_target_bin, ((0, 0), (0, hp_ - H), (0, 0)))
    bin_p = bin_p.reshape(B * hp_, 1, W)

    out_t, tgt_t, pooled_t = pl.pallas_call(
        _fused_kernel,
        grid=(B, nh),
        in_specs=[
            pl.BlockSpec((1, blk, _NBINS), lambda b, n: (b, n, 0)),
            pl.BlockSpec((1, W * _POOL, H * _POOL), lambda b, n: (b, 0, 0)),
            pl.BlockSpec((1, _HB, C, W), lambda b, n: (b, n, 0, 0)),
            pl.BlockSpec((_HB, 1, W), lambda b, n: (b * nh + n, 0, 0)),
        ],
        out_specs=[
            pl.BlockSpec((1, blk, C, _D), lambda b, n: (b, n, 0, 0)),
            pl.BlockSpec((1, blk, C, _D), lambda b, n: (b, n, 0, 0)),
            pl.BlockSpec((1, W, H), lambda b, n: (b, 0, 0)),
        ],
        out_shape=[
            jax.ShapeDtypeStruct((B, N, C, _D), jnp.float32),
            jax.ShapeDtypeStruct((B, N, C, _D), jnp.float32),
            jax.ShapeDtypeStruct((B, W, H), jnp.float32),
        ],
        compiler_params=pltpu.CompilerParams(
            dimension_semantics=("parallel", "arbitrary"),
            vmem_limit_bytes=56 * 1024 * 1024,
        ),
        name="depth_ffn_fused",
    )(logits_t, dm_t, img_n, bin_p)

    frustum = out_t.reshape(B, H, W, C, _D).transpose(0, 3, 4, 1, 2)
    frustum_tgt = tgt_t.reshape(B, H, W, C, _D).transpose(0, 3, 4, 1, 2)
    pooled = pooled_t.transpose(0, 2, 1)
    return frustum, frustum_tgt, pooled


# speedup vs baseline: 1.2534x; 1.0116x over previous
import jax
import jax.numpy as jnp
from jax.experimental import pallas as pl
from jax.experimental.pallas import tpu as pltpu

_D = 120
_NBINS = 121
_POOL = 8
_HB = 2


def _fused_kernel(logits_ref, dm_ref, img_ref, bin_ref,
                  out_ref, tgt_ref, pooled_ref):
    n = pl.program_id(1)

    @pl.when(n == pl.num_programs(1) - 1)
    def _pool():
        dm = dm_ref[0]
        ws, hs = dm.shape
        h, w = hs // _POOL, ws // _POOL
        ra = jax.lax.broadcasted_iota(jnp.int32, (w, ws), 0)
        ca = jax.lax.broadcasted_iota(jnp.int32, (w, ws), 1)
        pool_l = (ca // _POOL == ra).astype(jnp.float32)
        rb = jax.lax.broadcasted_iota(jnp.int32, (hs, h), 0)
        cb = jax.lax.broadcasted_iota(jnp.int32, (hs, h), 1)
        pool_r = (rb // _POOL == cb).astype(jnp.float32)
        hp = jax.lax.Precision.HIGHEST
        val = jnp.dot(
            jnp.dot(pool_l, dm, precision=hp,
                    preferred_element_type=jnp.float32),
            pool_r, precision=hp, preferred_element_type=jnp.float32)
        nz = (dm != 0.0).astype(jnp.float32)
        cnt = jnp.dot(
            jnp.dot(pool_l, nz, preferred_element_type=jnp.float32),
            pool_r, preferred_element_type=jnp.float32)
        inv = 1.0 / (_POOL * _POOL)
        pooled_ref[0] = (val * inv) / (cnt * inv + 1e-10)

    x = logits_ref[0]
    m = jnp.max(x, axis=-1, keepdims=True)
    e = jnp.exp(x - m)
    s = jnp.sum(e, axis=-1, keepdims=True)
    pv = (e / s)[:, :_D]

    img = img_ref[0]
    hb, c, w = img.shape
    p = hb * w
    img_parts = []
    bin_parts = []
    for h in range(hb):
        i_t = jnp.transpose(img[h])
        img_parts.append(jax.lax.broadcast_in_dim(i_t, (w, c, _D), (0, 1)))
        bin_parts.append(jnp.transpose(bin_ref[h]))
    img_b = jnp.concatenate(img_parts, axis=0)
    pv_b = jax.lax.broadcast_in_dim(pv, (p, c, _D), (0, 2))
    out_ref[0] = img_b * pv_b
    bvt = jnp.concatenate(bin_parts, axis=0)
    dd = jax.lax.broadcasted_iota(jnp.int32, (p, _D), 1)
    mask = dd == bvt
    mask_b = jax.lax.broadcast_in_dim(mask, (p, c, _D), (0, 2))
    tgt_ref[0] = jnp.where(mask_b, img_b, 0.0)


def kernel(image_features, depth_logits, depth_maps, depth_target_bin):
    B, C, H, W = image_features.shape
    N = H * W
    blk = _HB * W
    nh = -(-H // _HB)
    hp_ = nh * _HB

    logits_t = depth_logits.reshape(B, _NBINS, N).transpose(0, 2, 1)
    dm_t = depth_maps.transpose(0, 2, 1)
    img_n = image_features.transpose(0, 2, 1, 3)
    bin_p = jnp.pad(depth_target_bin, ((0, 0), (0, hp_ - H), (0, 0)))
    bin_p = bin_p.reshape(B * hp_, 1, W)

    out_t, tgt_t, pooled_t = pl.pallas_call(
        _fused_kernel,
        grid=(B, nh),
        in_specs=[
            pl.BlockSpec((1, blk, _NBINS), lambda b, n: (b, n, 0)),
            pl.BlockSpec((1, W * _POOL, H * _POOL), lambda b, n: (b, 0, 0)),
            pl.BlockSpec((1, _HB, C, W), lambda b, n: (b, n, 0, 0)),
            pl.BlockSpec((_HB, 1, W), lambda b, n: (b * nh + n, 0, 0)),
        ],
        out_specs=[
            pl.BlockSpec((1, blk, C, _D), lambda b, n: (b, n, 0, 0)),
            pl.BlockSpec((1, blk, C, _D), lambda b, n: (b, n, 0, 0)),
            pl.BlockSpec((1, W, H), lambda b, n: (b, 0, 0)),
        ],
        out_shape=[
            jax.ShapeDtypeStruct((B, N, C, _D), jnp.float32),
            jax.ShapeDtypeStruct((B, N, C, _D), jnp.float32),
            jax.ShapeDtypeStruct((B, W, H), jnp.float32),
        ],
        compiler_params=pltpu.CompilerParams(
            dimension_semantics=("parallel", "arbitrary"),
            vmem_limit_bytes=56 * 1024 * 1024,
        ),
        name="depth_ffn_fused",
    )(logits_t, dm_t, img_n, bin_p)

    frustum = out_t.reshape(B, H, W, C, _D).transpose(0, 3, 4, 1, 2)
    frustum_tgt = tgt_t.reshape(B, H, W, C, _D).transpose(0, 3, 4, 1, 2)
    pooled = pooled_t.transpose(0, 2, 1)
    return frustum, frustum_tgt, pooled
